# baseline (speedup 1.0000x reference)
import functools

import jax
import jax.numpy as jnp
from jax import lax
from jax.experimental import pallas as pl
from jax.experimental.pallas import tpu as pltpu
from jax.experimental.pallas import tpu_sc as plsc

_GAMMA = 0.1
_B = 64
_NW = 32
_ROW_BLK = 1024
_N_PAD = 10240


def _lane_bcast(v16, lane):
    idx = jnp.full((16, 1), lane, jnp.int32)
    return lax.gather(
        v16, idx,
        lax.GatherDimensionNumbers(
            offset_dims=(), collapsed_slice_dims=(0,), start_index_map=(0,)),
        slice_sizes=(1,),
        mode=lax.GatherScatterMode.PROMISE_IN_BOUNDS)


_CH = 4


def _make_sc_spmm(n, h, ba_f, nb_f, ba_s, nb_s, nb_max):
    rpt = n // 16
    assert nb_f % _CH == 0 and nb_s % _CH == 0
    ngrp = _B // 16
    mesh = plsc.VectorSubcoreMesh(
        core_axis_name="c", subcore_axis_name="s", num_cores=2,
        num_subcores=16)

    @functools.partial(
        pl.kernel,
        out_type=jax.ShapeDtypeStruct((2, n, h), jnp.float32),
        mesh=mesh,
        scratch_types=[
            pltpu.VMEM((n,), jnp.float32),
            pltpu.VMEM((2, _CH, 3, _B), jnp.int32),
            pltpu.VMEM((_B, h), jnp.float32),
            pltpu.VMEM((_B, h), jnp.float32),
            pltpu.VMEM((_B, h), jnp.float32),
            pltpu.VMEM((_B, h), jnp.float32),
            pltpu.VMEM((8, h), jnp.float32),
            pltpu.VMEM_SHARED((n, h), jnp.float32),
            pltpu.SemaphoreType.DMA,
            pltpu.SemaphoreType.DMA,
            pltpu.SemaphoreType.DMA,
            pltpu.SemaphoreType.DMA,
            pltpu.SemaphoreType.DMA,
            pltpu.SemaphoreType.DMA,
            pltpu.SemaphoreType.DMA,
            pltpu.SemaphoreType.DMA,
            pltpu.SemaphoreType.DMA,
        ],
        compiler_params=pltpu.CompilerParams(
            needs_layout_passes=False, use_tc_tiling_on_sc=False),
    )
    def spmm_kernel(s_hbm, tab_hbm, comb_hbm, out_hbm,
                    s_v, comb_v, rows0, rows1, rows2, rows3, z_v, acc,
                    semg0, semg1, semg2, semg3,
                    sems0, sems1, sems2, sems3, semc):
        c = lax.axis_index("c")
        s = lax.axis_index("s")
        wid = c * 16 + s
        r0 = s * rpt
        nchunks = jnp.where(c == 0, nb_f // _CH, nb_s // _CH)
        blocks_adj = jnp.where(c == 0, ba_f, ba_s)

        zero = jnp.zeros((16,), jnp.float32)
        for r in range(8):
            for k in range(h // 16):
                z_v[r, pl.ds(k * 16, 16)] = zero

        def zblk(j, carry):
            pltpu.sync_copy(z_v, acc.at[pl.ds(r0 + j * 8, 8)])
            return carry

        lax.fori_loop(0, rpt // 8, zblk, 0)

        pltpu.sync_copy(s_hbm, s_v)
        pltpu.sync_copy(comb_hbm.at[wid, pl.ds(0, _CH)], comb_v.at[0])
        plsc.subcore_barrier()

        def stage_start(q):
            pltpu.async_copy(comb_hbm.at[wid, pl.ds(q * _CH, _CH)],
                             comb_v.at[q % 2], semc)

        def stage_wait(q):
            pltpu.make_async_copy(comb_hbm.at[wid, pl.ds(q * _CH, _CH)],
                                  comb_v.at[q % 2], semc).wait()

        def gather_start(qp, b, rows, semg):
            pltpu.async_copy(tab_hbm.at[comb_v.at[qp, b, 0]], rows, semg)

        def gather_wait(qp, b, rows, semg):
            pltpu.make_async_copy(tab_hbm.at[comb_v.at[qp, b, 0]], rows,
                                  semg).wait()

        def scatter_start(qp, b, rows, sems):
            pltpu.async_copy(rows, acc.at[comb_v.at[qp, b, 1]], sems,
                             add=True)

        def scatter_wait(qp, b, rows, sems):
            pltpu.make_async_copy(rows, acc.at[comb_v.at[qp, b, 1]],
                                  sems).wait()

        def scale(i, qp, b, rows):
            def grp(g, carry):
                gs = pl.ds(g * 16, 16)
                dst16 = comb_v[qp, b, 1, gs]
                w16 = plsc.bitcast(comb_v[qp, b, 2, gs], jnp.float32)
                sg = plsc.load_gather(s_v, [dst16])
                gate = jnp.where(i >= blocks_adj, 1.0 - sg, sg)
                ws16 = w16 * gate
                for lane in range(16):
                    wb = _lane_bcast(ws16, lane)
                    e = g * 16 + lane
                    for k in range(h // 16):
                        cs = pl.ds(k * 16, 16)
                        rows[e, cs] = rows[e, cs] * wb
                return carry

            lax.fori_loop(0, ngrp, grp, 0)

        bufs = [(rows0, semg0, sems0), (rows1, semg1, sems1),
                (rows2, semg2, sems2), (rows3, semg3, sems3)]
        gather_start(0, 0, rows0, semg0)

        def body(q, carry):
            qp = q % 2

            for b in range(_CH):
                i = q * _CH + b
                rows, semg, sems = bufs[b]
                rn, semg_n, sems_n = bufs[(b + 1) % _CH]

                if b == _CH - 1:
                    scatter_wait(qp, 0, rn, sems_n)
                else:
                    @pl.when(q >= 1)
                    def _():
                        scatter_wait(1 - qp, b + 1, rn, sems_n)

                if b == 2:
                    @pl.when(q + 1 < nchunks)
                    def _():
                        stage_start(q + 1)

                if b < _CH - 1:
                    gather_start(qp, b + 1, rn, semg_n)
                else:
                    @pl.when(q + 1 < nchunks)
                    def _():
                        stage_wait(q + 1)
                        gather_start(1 - qp, 0, rn, semg_n)

                gather_wait(qp, b, rows, semg)
                scale(i, qp, b, rows)
                scatter_start(qp, b, rows, sems)
            return carry

        lax.fori_loop(0, nchunks, body, 0)
        lq = lax.rem(nchunks - 1, 2)
        for b in range(1, _CH):
            rows_l, _, sems_l = bufs[b]
            scatter_wait(lq, b, rows_l, sems_l)
        plsc.subcore_barrier()
        pltpu.sync_copy(acc.at[pl.ds(r0, rpt)],
                        out_hbm.at[c, pl.ds(r0, rpt)])

    return spmm_kernel


def _sigmoid(z):
    return 1.0 / (1.0 + jnp.exp(-z))


def _tc_layer1(fea, W_in, W_in_self, b_in, scores0, Dk0, bias0, Dbias0):
    n, f = fea.shape
    hh = W_in.shape[1]
    grid = (n // _ROW_BLK,)

    def body(f_ref, win_ref, wins_ref, bin_ref, sc_ref, dk_ref, b0_ref,
             db_ref, s_out, d_out, sig_out):
        x = f_ref[...]
        S = jnp.dot(x, win_ref[...], preferred_element_type=jnp.float32)
        sid = _sigmoid(
            jnp.dot(x, sc_ref[...], preferred_element_type=jnp.float32)
            + b0_ref[...])
        dk = jnp.dot(x, dk_ref[...], preferred_element_type=jnp.float32) \
            + db_ref[...]
        self_t = jnp.dot(x, wins_ref[...],
                         preferred_element_type=jnp.float32) + bin_ref[...]
        D = self_t + _GAMMA * dk * (S + self_t)
        s_out[...] = S
        d_out[...] = D
        sig_out[...] = sid

    return pl.pallas_call(
        body,
        grid=grid,
        in_specs=[
            pl.BlockSpec((_ROW_BLK, f), lambda i: (i, 0)),
            pl.BlockSpec((f, hh), lambda i: (0, 0)),
            pl.BlockSpec((f, hh), lambda i: (0, 0)),
            pl.BlockSpec((hh,), lambda i: (0,)),
            pl.BlockSpec((f, 1), lambda i: (0, 0)),
            pl.BlockSpec((f, 1), lambda i: (0, 0)),
            pl.BlockSpec((1,), lambda i: (0,)),
            pl.BlockSpec((1,), lambda i: (0,)),
        ],
        out_specs=[
            pl.BlockSpec((_ROW_BLK, hh), lambda i: (i, 0)),
            pl.BlockSpec((_ROW_BLK, hh), lambda i: (i, 0)),
            pl.BlockSpec((_ROW_BLK, 1), lambda i: (i, 0)),
        ],
        out_shape=[
            jax.ShapeDtypeStruct((n, hh), jnp.float32),
            jax.ShapeDtypeStruct((n, hh), jnp.float32),
            jax.ShapeDtypeStruct((n, 1), jnp.float32),
        ],
    )(fea, W_in, W_in_self, b_in, scores0, Dk0, bias0, Dbias0)


def _tc_layer2(parts, dense, W_out, W_out_self, b_out, scores0, Dk0, bias0,
               Dbias0):
    _, n, hh = parts.shape
    cc = W_out.shape[1]
    grid = (n // _ROW_BLK,)

    def body(p_ref, d_ref, wo_ref, wos_ref, bo_ref, sc_ref, dk_ref, b0_ref,
             db_ref, s_out, d_out, sig_out):
        x = p_ref[0] + p_ref[1] + d_ref[...]
        S = jnp.dot(x, wo_ref[...], preferred_element_type=jnp.float32)
        sid = _sigmoid(
            jnp.dot(x, sc_ref[...], preferred_element_type=jnp.float32)
            + b0_ref[...])
        dk = jnp.dot(x, dk_ref[...], preferred_element_type=jnp.float32) \
            + db_ref[...]
        self_t = jnp.dot(x, wos_ref[...],
                         preferred_element_type=jnp.float32) + bo_ref[...]
        D = self_t + _GAMMA * dk * (S + self_t)
        s_out[...] = S
        d_out[...] = D
        sig_out[...] = sid

    return pl.pallas_call(
        body,
        grid=grid,
        in_specs=[
            pl.BlockSpec((2, _ROW_BLK, hh), lambda i: (0, i, 0)),
            pl.BlockSpec((_ROW_BLK, hh), lambda i: (i, 0)),
            pl.BlockSpec((hh, cc), lambda i: (0, 0)),
            pl.BlockSpec((hh, cc), lambda i: (0, 0)),
            pl.BlockSpec((cc,), lambda i: (0,)),
            pl.BlockSpec((hh, 1), lambda i: (0, 0)),
            pl.BlockSpec((hh, 1), lambda i: (0, 0)),
            pl.BlockSpec((1,), lambda i: (0,)),
            pl.BlockSpec((1,), lambda i: (0,)),
        ],
        out_specs=[
            pl.BlockSpec((_ROW_BLK, cc), lambda i: (i, 0)),
            pl.BlockSpec((_ROW_BLK, cc), lambda i: (i, 0)),
            pl.BlockSpec((_ROW_BLK, 1), lambda i: (i, 0)),
        ],
        out_shape=[
            jax.ShapeDtypeStruct((n, cc), jnp.float32),
            jax.ShapeDtypeStruct((n, cc), jnp.float32),
            jax.ShapeDtypeStruct((n, 1), jnp.float32),
        ],
    )(parts, dense, W_out, W_out_self, b_out, scores0, Dk0, bias0, Dbias0)


def _tc_final(parts, dense):
    _, n, cc = parts.shape
    grid = (n // _ROW_BLK,)

    def body(p_ref, d_ref, o_ref):
        z = p_ref[0] + p_ref[1] + d_ref[...]
        m = jnp.max(z, axis=1, keepdims=True)
        zm = z - m
        o_ref[...] = zm - jnp.log(jnp.sum(jnp.exp(zm), axis=1, keepdims=True))

    return pl.pallas_call(
        body,
        grid=grid,
        in_specs=[pl.BlockSpec((2, _ROW_BLK, cc), lambda i: (0, i, 0)),
                  pl.BlockSpec((_ROW_BLK, cc), lambda i: (i, 0))],
        out_specs=pl.BlockSpec((_ROW_BLK, cc), lambda i: (i, 0)),
        out_shape=jax.ShapeDtypeStruct((n, cc), jnp.float32),
    )(parts, dense)


_FAST_SHARE = 0.55


def _pack_rows(index, weight, lo, hi):
    e = hi - lo
    unit = 16 * _B
    epad = ((e + unit - 1) // unit) * unit
    pad = epad - e
    src = jnp.concatenate([index[0][lo:hi], jnp.zeros((pad,), jnp.int32)])
    dst = jnp.concatenate([index[1][lo:hi], jnp.zeros((pad,), jnp.int32)])
    w = jnp.concatenate([weight[lo:hi], jnp.zeros((pad,), jnp.float32)])
    wi = lax.bitcast_convert_type(w, jnp.int32)
    nbw = epad // unit
    comb = jnp.stack([x.reshape(16, nbw, _B) for x in (src, dst, wi)],
                     axis=2)
    return comb, nbw


def _split_pack(index, weight):
    e = weight.shape[0]
    unit = 16 * _B
    ef = int(e * _FAST_SHARE) // unit * unit
    cf, bf = _pack_rows(index, weight, 0, ef)
    cs, bs = _pack_rows(index, weight, ef, e)
    return cf, bf, cs, bs


def _even4(x):
    return ((x + _CH - 1) // _CH) * _CH


def kernel(fea, adj_index, adj_weight, adj_knn_index, adj_knn_weight,
           W_in, W_in_self, b_in, W_out, W_out_self, b_out,
           scores0, bias0, Dk0, Dbias0):
    n_real = fea.shape[0]
    n = _N_PAD
    fea = jnp.pad(fea, ((0, n - n_real), (0, 0)))
    hh = W_in.shape[1]
    cc = W_out.shape[1]

    caf, baf, cas, bas = _split_pack(adj_index, adj_weight)
    ckf, bkf, cks, bks = _split_pack(adj_knn_index, adj_knn_weight)
    nb_f, nb_s = _even4(baf + bkf), _even4(bas + bks)
    nb_max = max(nb_f, nb_s)

    def _rows(ca, ck, used):
        pads = ([jnp.zeros((16, nb_max - used, 3, _B), jnp.int32)]
                if nb_max > used else [])
        return jnp.concatenate([ca, ck] + pads, axis=1)

    comb = jnp.concatenate(
        [_rows(caf, ckf, baf + bkf), _rows(cas, cks, bas + bks)], axis=0)

    S1, D1, sig1 = _tc_layer1(fea, W_in, W_in_self, b_in, scores0, Dk0,
                              bias0, Dbias0)
    sc1 = _make_sc_spmm(n, hh, baf, nb_f, bas, nb_s, nb_max)
    parts1 = sc1(sig1.reshape(n), S1, comb)

    S2, D2, sig2 = _tc_layer2(parts1, D1, W_out, W_out_self, b_out, scores0,
                              Dk0, bias0, Dbias0)
    sc2 = _make_sc_spmm(n, cc, baf, nb_f, bas, nb_s, nb_max)
    parts2 = sc2(sig2.reshape(n), S2, comb)

    return _tc_final(parts2, D2)[:n_real]


# speedup vs baseline: 1.0940x; 1.0940x over previous
import functools

import jax
import jax.numpy as jnp
from jax import lax
from jax.experimental import pallas as pl
from jax.experimental.pallas import tpu as pltpu
from jax.experimental.pallas import tpu_sc as plsc

_GAMMA = 0.1
_B = 64
_NW = 32
_ROW_BLK = 1024
_N_PAD = 10240


def _lane_bcast(v16, lane):
    idx = jnp.full((16, 1), lane, jnp.int32)
    return lax.gather(
        v16, idx,
        lax.GatherDimensionNumbers(
            offset_dims=(), collapsed_slice_dims=(0,), start_index_map=(0,)),
        slice_sizes=(1,),
        mode=lax.GatherScatterMode.PROMISE_IN_BOUNDS)


_CH = 4


def _make_sc_spmm(n, h, ba_f, nb_f, ba_s, nb_s, nb_max):
    rpt = n // 16
    assert nb_f % _CH == 0 and nb_s % _CH == 0
    ngrp = _B // 16
    mesh = plsc.VectorSubcoreMesh(
        core_axis_name="c", subcore_axis_name="s", num_cores=2,
        num_subcores=16)

    @functools.partial(
        pl.kernel,
        out_type=jax.ShapeDtypeStruct((2, n, h), jnp.float32),
        mesh=mesh,
        scratch_types=[
            pltpu.VMEM((n,), jnp.float32),
            pltpu.VMEM((2, _CH, 3, _B), jnp.int32),
            pltpu.VMEM((_B, h), jnp.float32),
            pltpu.VMEM((_B, h), jnp.float32),
            pltpu.VMEM((_B, h), jnp.float32),
            pltpu.VMEM((_B, h), jnp.float32),
            pltpu.VMEM((8, h), jnp.float32),
            pltpu.VMEM_SHARED((n, h), jnp.float32),
            pltpu.SemaphoreType.DMA,
            pltpu.SemaphoreType.DMA,
            pltpu.SemaphoreType.DMA,
            pltpu.SemaphoreType.DMA,
            pltpu.SemaphoreType.DMA,
            pltpu.SemaphoreType.DMA,
            pltpu.SemaphoreType.DMA,
            pltpu.SemaphoreType.DMA,
            pltpu.SemaphoreType.DMA,
        ],
        compiler_params=pltpu.CompilerParams(
            needs_layout_passes=False, use_tc_tiling_on_sc=False),
    )
    def spmm_kernel(s_hbm, tab_hbm, comb_hbm, out_hbm,
                    s_v, comb_v, rows0, rows1, rows2, rows3, z_v, acc,
                    semg0, semg1, semg2, semg3,
                    sems0, sems1, sems2, sems3, semc):
        c = lax.axis_index("c")
        s = lax.axis_index("s")
        wid = c * 16 + s
        r0 = s * rpt
        nchunks = (nb_f // _CH if nb_f == nb_s
                   else jnp.where(c == 0, nb_f // _CH, nb_s // _CH))
        blocks_adj = (ba_f if ba_f == ba_s
                      else jnp.where(c == 0, ba_f, ba_s))

        zero = jnp.zeros((16,), jnp.float32)
        for r in range(8):
            for k in range(h // 16):
                z_v[r, pl.ds(k * 16, 16)] = zero

        def zblk(j, carry):
            pltpu.sync_copy(z_v, acc.at[pl.ds(r0 + j * 8, 8)])
            return carry

        lax.fori_loop(0, rpt // 8, zblk, 0)

        pltpu.sync_copy(s_hbm, s_v)
        pltpu.sync_copy(comb_hbm.at[wid, pl.ds(0, _CH)], comb_v.at[0])
        plsc.subcore_barrier()

        def stage_start(q):
            pltpu.async_copy(comb_hbm.at[wid, pl.ds(q * _CH, _CH)],
                             comb_v.at[q % 2], semc)

        def stage_wait(q):
            pltpu.make_async_copy(comb_hbm.at[wid, pl.ds(q * _CH, _CH)],
                                  comb_v.at[q % 2], semc).wait()

        def gather_start(qp, b, rows, semg):
            pltpu.async_copy(tab_hbm.at[comb_v.at[qp, b, 0]], rows, semg)

        def gather_wait(qp, b, rows, semg):
            pltpu.make_async_copy(tab_hbm.at[comb_v.at[qp, b, 0]], rows,
                                  semg).wait()

        def scatter_start(qp, b, rows, sems):
            pltpu.async_copy(rows, acc.at[comb_v.at[qp, b, 1]], sems,
                             add=True)

        def scatter_wait(qp, b, rows, sems):
            pltpu.make_async_copy(rows, acc.at[comb_v.at[qp, b, 1]],
                                  sems).wait()

        def scale(i, qp, b, rows):
            def grp(g, carry):
                gs = pl.ds(g * 16, 16)
                dst16 = comb_v[qp, b, 1, gs]
                w16 = plsc.bitcast(comb_v[qp, b, 2, gs], jnp.float32)
                sg = plsc.load_gather(s_v, [dst16])
                gate = jnp.where(i >= blocks_adj, 1.0 - sg, sg)
                ws16 = w16 * gate
                for lane in range(16):
                    wb = _lane_bcast(ws16, lane)
                    e = g * 16 + lane
                    for k in range(h // 16):
                        cs = pl.ds(k * 16, 16)
                        rows[e, cs] = rows[e, cs] * wb
                return carry

            lax.fori_loop(0, ngrp, grp, 0)

        bufs = [(rows0, semg0, sems0), (rows1, semg1, sems1),
                (rows2, semg2, sems2), (rows3, semg3, sems3)]
        gather_start(0, 0, rows0, semg0)

        def body(q, carry):
            qp = q % 2

            for b in range(_CH):
                i = q * _CH + b
                rows, semg, sems = bufs[b]
                rn, semg_n, sems_n = bufs[(b + 1) % _CH]

                if b == _CH - 1:
                    scatter_wait(qp, 0, rn, sems_n)
                else:
                    @pl.when(q >= 1)
                    def _():
                        scatter_wait(1 - qp, b + 1, rn, sems_n)

                if b == 2:
                    @pl.when(q + 1 < nchunks)
                    def _():
                        stage_start(q + 1)

                if b < _CH - 1:
                    gather_start(qp, b + 1, rn, semg_n)
                else:
                    @pl.when(q + 1 < nchunks)
                    def _():
                        stage_wait(q + 1)
                        gather_start(1 - qp, 0, rn, semg_n)

                gather_wait(qp, b, rows, semg)
                scale(i, qp, b, rows)
                scatter_start(qp, b, rows, sems)
            return carry

        lax.fori_loop(0, nchunks, body, 0)
        lq = (nchunks - 1) % 2
        for b in range(1, _CH):
            rows_l, _, sems_l = bufs[b]
            scatter_wait(lq, b, rows_l, sems_l)
        plsc.subcore_barrier()
        pltpu.sync_copy(acc.at[pl.ds(r0, rpt)],
                        out_hbm.at[c, pl.ds(r0, rpt)])

    return spmm_kernel


def _sigmoid(z):
    return 1.0 / (1.0 + jnp.exp(-z))


def _tc_layer1(fea, W_in, W_in_self, b_in, scores0, Dk0, bias0, Dbias0):
    n, f = fea.shape
    hh = W_in.shape[1]
    grid = (n // _ROW_BLK,)

    def body(f_ref, win_ref, wins_ref, bin_ref, sc_ref, dk_ref, b0_ref,
             db_ref, s_out, d_out, sig_out):
        x = f_ref[...]
        S = jnp.dot(x, win_ref[...], preferred_element_type=jnp.float32)
        sid = _sigmoid(
            jnp.dot(x, sc_ref[...], preferred_element_type=jnp.float32)
            + b0_ref[...])
        dk = jnp.dot(x, dk_ref[...], preferred_element_type=jnp.float32) \
            + db_ref[...]
        self_t = jnp.dot(x, wins_ref[...],
                         preferred_element_type=jnp.float32) + bin_ref[...]
        D = self_t + _GAMMA * dk * (S + self_t)
        s_out[...] = S
        d_out[...] = D
        sig_out[...] = sid

    return pl.pallas_call(
        body,
        grid=grid,
        in_specs=[
            pl.BlockSpec((_ROW_BLK, f), lambda i: (i, 0)),
            pl.BlockSpec((f, hh), lambda i: (0, 0)),
            pl.BlockSpec((f, hh), lambda i: (0, 0)),
            pl.BlockSpec((hh,), lambda i: (0,)),
            pl.BlockSpec((f, 1), lambda i: (0, 0)),
            pl.BlockSpec((f, 1), lambda i: (0, 0)),
            pl.BlockSpec((1,), lambda i: (0,)),
            pl.BlockSpec((1,), lambda i: (0,)),
        ],
        out_specs=[
            pl.BlockSpec((_ROW_BLK, hh), lambda i: (i, 0)),
            pl.BlockSpec((_ROW_BLK, hh), lambda i: (i, 0)),
            pl.BlockSpec((_ROW_BLK, 1), lambda i: (i, 0)),
        ],
        out_shape=[
            jax.ShapeDtypeStruct((n, hh), jnp.float32),
            jax.ShapeDtypeStruct((n, hh), jnp.float32),
            jax.ShapeDtypeStruct((n, 1), jnp.float32),
        ],
    )(fea, W_in, W_in_self, b_in, scores0, Dk0, bias0, Dbias0)


def _tc_layer2(parts, dense, W_out, W_out_self, b_out, scores0, Dk0, bias0,
               Dbias0):
    _, n, hh = parts.shape
    cc = W_out.shape[1]
    grid = (n // _ROW_BLK,)

    def body(p_ref, d_ref, wo_ref, wos_ref, bo_ref, sc_ref, dk_ref, b0_ref,
             db_ref, s_out, d_out, sig_out):
        x = p_ref[0] + p_ref[1] + d_ref[...]
        S = jnp.dot(x, wo_ref[...], preferred_element_type=jnp.float32)
        sid = _sigmoid(
            jnp.dot(x, sc_ref[...], preferred_element_type=jnp.float32)
            + b0_ref[...])
        dk = jnp.dot(x, dk_ref[...], preferred_element_type=jnp.float32) \
            + db_ref[...]
        self_t = jnp.dot(x, wos_ref[...],
                         preferred_element_type=jnp.float32) + bo_ref[...]
        D = self_t + _GAMMA * dk * (S + self_t)
        s_out[...] = S
        d_out[...] = D
        sig_out[...] = sid

    return pl.pallas_call(
        body,
        grid=grid,
        in_specs=[
            pl.BlockSpec((2, _ROW_BLK, hh), lambda i: (0, i, 0)),
            pl.BlockSpec((_ROW_BLK, hh), lambda i: (i, 0)),
            pl.BlockSpec((hh, cc), lambda i: (0, 0)),
            pl.BlockSpec((hh, cc), lambda i: (0, 0)),
            pl.BlockSpec((cc,), lambda i: (0,)),
            pl.BlockSpec((hh, 1), lambda i: (0, 0)),
            pl.BlockSpec((hh, 1), lambda i: (0, 0)),
            pl.BlockSpec((1,), lambda i: (0,)),
            pl.BlockSpec((1,), lambda i: (0,)),
        ],
        out_specs=[
            pl.BlockSpec((_ROW_BLK, cc), lambda i: (i, 0)),
            pl.BlockSpec((_ROW_BLK, cc), lambda i: (i, 0)),
            pl.BlockSpec((_ROW_BLK, 1), lambda i: (i, 0)),
        ],
        out_shape=[
            jax.ShapeDtypeStruct((n, cc), jnp.float32),
            jax.ShapeDtypeStruct((n, cc), jnp.float32),
            jax.ShapeDtypeStruct((n, 1), jnp.float32),
        ],
    )(parts, dense, W_out, W_out_self, b_out, scores0, Dk0, bias0, Dbias0)


def _tc_final(parts, dense):
    _, n, cc = parts.shape
    grid = (n // _ROW_BLK,)

    def body(p_ref, d_ref, o_ref):
        z = p_ref[0] + p_ref[1] + d_ref[...]
        m = jnp.max(z, axis=1, keepdims=True)
        zm = z - m
        o_ref[...] = zm - jnp.log(jnp.sum(jnp.exp(zm), axis=1, keepdims=True))

    return pl.pallas_call(
        body,
        grid=grid,
        in_specs=[pl.BlockSpec((2, _ROW_BLK, cc), lambda i: (0, i, 0)),
                  pl.BlockSpec((_ROW_BLK, cc), lambda i: (i, 0))],
        out_specs=pl.BlockSpec((_ROW_BLK, cc), lambda i: (i, 0)),
        out_shape=jax.ShapeDtypeStruct((n, cc), jnp.float32),
    )(parts, dense)


_FAST_SHARE = 0.5


def _pack_rows(index, weight, lo, hi):
    e = hi - lo
    unit = 16 * _B
    epad = ((e + unit - 1) // unit) * unit
    pad = epad - e
    src = jnp.concatenate([index[0][lo:hi], jnp.zeros((pad,), jnp.int32)])
    dst = jnp.concatenate([index[1][lo:hi], jnp.zeros((pad,), jnp.int32)])
    w = jnp.concatenate([weight[lo:hi], jnp.zeros((pad,), jnp.float32)])
    wi = lax.bitcast_convert_type(w, jnp.int32)
    nbw = epad // unit
    comb = jnp.stack([x.reshape(16, nbw, _B) for x in (src, dst, wi)],
                     axis=2)
    return comb, nbw


def _split_pack(index, weight):
    e = weight.shape[0]
    unit = 16 * _B
    ef = int(e * _FAST_SHARE) // unit * unit
    cf, bf = _pack_rows(index, weight, 0, ef)
    cs, bs = _pack_rows(index, weight, ef, e)
    return cf, bf, cs, bs


def _even4(x):
    return ((x + _CH - 1) // _CH) * _CH


def kernel(fea, adj_index, adj_weight, adj_knn_index, adj_knn_weight,
           W_in, W_in_self, b_in, W_out, W_out_self, b_out,
           scores0, bias0, Dk0, Dbias0):
    n_real = fea.shape[0]
    n = _N_PAD
    fea = jnp.pad(fea, ((0, n - n_real), (0, 0)))
    hh = W_in.shape[1]
    cc = W_out.shape[1]

    caf, baf, cas, bas = _split_pack(adj_index, adj_weight)
    ckf, bkf, cks, bks = _split_pack(adj_knn_index, adj_knn_weight)
    nb_f, nb_s = _even4(baf + bkf), _even4(bas + bks)
    nb_max = max(nb_f, nb_s)

    def _rows(ca, ck, used):
        pads = ([jnp.zeros((16, nb_max - used, 3, _B), jnp.int32)]
                if nb_max > used else [])
        return jnp.concatenate([ca, ck] + pads, axis=1)

    comb = jnp.concatenate(
        [_rows(caf, ckf, baf + bkf), _rows(cas, cks, bas + bks)], axis=0)

    S1, D1, sig1 = _tc_layer1(fea, W_in, W_in_self, b_in, scores0, Dk0,
                              bias0, Dbias0)
    sc1 = _make_sc_spmm(n, hh, baf, nb_f, bas, nb_s, nb_max)
    parts1 = sc1(sig1.reshape(n), S1, comb)

    S2, D2, sig2 = _tc_layer2(parts1, D1, W_out, W_out_self, b_out, scores0,
                              Dk0, bias0, Dbias0)
    sc2 = _make_sc_spmm(n, cc, baf, nb_f, bas, nb_s, nb_max)
    parts2 = sc2(sig2.reshape(n), S2, comb)

    return _tc_final(parts2, D2)[:n_real]


# speedup vs baseline: 1.1602x; 1.0605x over previous
import functools

import jax
import jax.numpy as jnp
from jax import lax
from jax.experimental import pallas as pl
from jax.experimental.pallas import tpu as pltpu
from jax.experimental.pallas import tpu_sc as plsc

_GAMMA = 0.1
_B = 64
_NW = 32
_ROW_BLK = 1024
_N_PAD = 10240


def _lane_bcast(v16, lane):
    idx = jnp.full((16, 1), lane, jnp.int32)
    return lax.gather(
        v16, idx,
        lax.GatherDimensionNumbers(
            offset_dims=(), collapsed_slice_dims=(0,), start_index_map=(0,)),
        slice_sizes=(1,),
        mode=lax.GatherScatterMode.PROMISE_IN_BOUNDS)


_CH = 4


def _make_sc_spmm(n, h, ba_f, nb_f, ba_s, nb_s, nb_max):
    rpt = n // 16
    assert nb_f % _CH == 0 and nb_s % _CH == 0
    ngrp = _B // 16
    mesh = plsc.VectorSubcoreMesh(
        core_axis_name="c", subcore_axis_name="s", num_cores=2,
        num_subcores=16)

    @functools.partial(
        pl.kernel,
        out_type=jax.ShapeDtypeStruct((2, n, h), jnp.float32),
        mesh=mesh,
        scratch_types=[
            pltpu.VMEM((n,), jnp.float32),
            pltpu.VMEM((2, _CH, 3, _B), jnp.int32),
            pltpu.VMEM((_B, h), jnp.float32),
            pltpu.VMEM((_B, h), jnp.float32),
            pltpu.VMEM((_B, h), jnp.float32),
            pltpu.VMEM((_B, h), jnp.float32),
            pltpu.VMEM((8, h), jnp.float32),
            pltpu.VMEM_SHARED((n, h), jnp.float32),
            pltpu.SemaphoreType.DMA,
            pltpu.SemaphoreType.DMA,
            pltpu.SemaphoreType.DMA,
            pltpu.SemaphoreType.DMA,
            pltpu.SemaphoreType.DMA,
            pltpu.SemaphoreType.DMA,
            pltpu.SemaphoreType.DMA,
            pltpu.SemaphoreType.DMA,
            pltpu.SemaphoreType.DMA,
        ],
        compiler_params=pltpu.CompilerParams(
            needs_layout_passes=False, use_tc_tiling_on_sc=False),
    )
    def spmm_kernel(s_hbm, tab_hbm, comb_hbm, out_hbm,
                    s_v, comb_v, rows0, rows1, rows2, rows3, z_v, acc,
                    semg0, semg1, semg2, semg3,
                    sems0, sems1, sems2, sems3, semc):
        c = lax.axis_index("c")
        s = lax.axis_index("s")
        wid = c * 16 + s
        r0 = s * rpt
        nchunks = (nb_f // _CH if nb_f == nb_s
                   else jnp.where(c == 0, nb_f // _CH, nb_s // _CH))
        blocks_adj = (ba_f if ba_f == ba_s
                      else jnp.where(c == 0, ba_f, ba_s))

        zero = jnp.zeros((16,), jnp.float32)
        for r in range(8):
            for k in range(h // 16):
                z_v[r, pl.ds(k * 16, 16)] = zero

        def zblk(j, carry):
            pltpu.sync_copy(z_v, acc.at[pl.ds(r0 + j * 8, 8)])
            return carry

        lax.fori_loop(0, rpt // 8, zblk, 0)

        pltpu.sync_copy(s_hbm, s_v)
        pltpu.sync_copy(comb_hbm.at[wid, pl.ds(0, _CH)], comb_v.at[0])
        plsc.subcore_barrier()

        def stage_start(q):
            pltpu.async_copy(comb_hbm.at[wid, pl.ds(q * _CH, _CH)],
                             comb_v.at[q % 2], semc)

        def stage_wait(q):
            pltpu.make_async_copy(comb_hbm.at[wid, pl.ds(q * _CH, _CH)],
                                  comb_v.at[q % 2], semc).wait()

        def gather_start(qp, b, rows, semg):
            pltpu.async_copy(tab_hbm.at[comb_v.at[qp, b, 0]], rows, semg)

        def gather_wait(qp, b, rows, semg):
            pltpu.make_async_copy(tab_hbm.at[comb_v.at[qp, b, 0]], rows,
                                  semg).wait()

        def scatter_start(qp, b, rows, sems):
            pltpu.async_copy(rows, acc.at[comb_v.at[qp, b, 1]], sems,
                             add=True)

        def scatter_wait(qp, b, rows, sems):
            pltpu.make_async_copy(rows, acc.at[comb_v.at[qp, b, 1]],
                                  sems).wait()

        def scale(i, qp, b, rows):
            def grp(g, carry):
                gs = pl.ds(g * 16, 16)
                dst16 = comb_v[qp, b, 1, gs]
                w16 = plsc.bitcast(comb_v[qp, b, 2, gs], jnp.float32)
                sg = plsc.load_gather(s_v, [dst16])
                gate = jnp.where(i >= blocks_adj, 1.0 - sg, sg)
                ws16 = w16 * gate
                for lane in range(16):
                    wb = _lane_bcast(ws16, lane)
                    e = g * 16 + lane
                    for k in range(h // 16):
                        cs = pl.ds(k * 16, 16)
                        rows[e, cs] = rows[e, cs] * wb
                return carry

            lax.fori_loop(0, ngrp, grp, 0)

        bufs = [(rows0, semg0, sems0), (rows1, semg1, sems1),
                (rows2, semg2, sems2), (rows3, semg3, sems3)]
        gather_start(0, 0, rows0, semg0)
        gather_start(0, 1, rows1, semg1)

        def body(q, carry):
            qp = q % 2

            for b in range(_CH):
                i = q * _CH + b
                rows, semg, sems = bufs[b]
                rn, semg_n, sems_n = bufs[(b + 2) % _CH]

                if b < 2:
                    @pl.when(q >= 1)
                    def _():
                        scatter_wait(1 - qp, b + 2, rn, sems_n)

                    if b == 1:
                        @pl.when(q + 1 < nchunks)
                        def _():
                            stage_start(q + 1)
                else:
                    scatter_wait(qp, b - 2, rn, sems_n)

                if b < 2:
                    gather_start(qp, b + 2, rn, semg_n)
                else:
                    @pl.when(q + 1 < nchunks)
                    def _():
                        if b == 2:
                            stage_wait(q + 1)
                        gather_start(1 - qp, b - 2, rn, semg_n)

                gather_wait(qp, b, rows, semg)
                scale(i, qp, b, rows)
                scatter_start(qp, b, rows, sems)
            return carry

        lax.fori_loop(0, nchunks, body, 0)
        lq = (nchunks - 1) % 2
        for b in range(2, _CH):
            rows_l, _, sems_l = bufs[b]
            scatter_wait(lq, b, rows_l, sems_l)
        plsc.subcore_barrier()
        pltpu.sync_copy(acc.at[pl.ds(r0, rpt)],
                        out_hbm.at[c, pl.ds(r0, rpt)])

    return spmm_kernel


def _sigmoid(z):
    return 1.0 / (1.0 + jnp.exp(-z))


def _tc_layer1(fea, W_in, W_in_self, b_in, scores0, Dk0, bias0, Dbias0):
    n, f = fea.shape
    hh = W_in.shape[1]
    grid = (n // _ROW_BLK,)

    def body(f_ref, win_ref, wins_ref, bin_ref, sc_ref, dk_ref, b0_ref,
             db_ref, s_out, d_out, sig_out):
        x = f_ref[...]
        S = jnp.dot(x, win_ref[...], preferred_element_type=jnp.float32)
        sid = _sigmoid(
            jnp.dot(x, sc_ref[...], preferred_element_type=jnp.float32)
            + b0_ref[...])
        dk = jnp.dot(x, dk_ref[...], preferred_element_type=jnp.float32) \
            + db_ref[...]
        self_t = jnp.dot(x, wins_ref[...],
                         preferred_element_type=jnp.float32) + bin_ref[...]
        D = self_t + _GAMMA * dk * (S + self_t)
        s_out[...] = S
        d_out[...] = D
        sig_out[...] = sid

    return pl.pallas_call(
        body,
        grid=grid,
        in_specs=[
            pl.BlockSpec((_ROW_BLK, f), lambda i: (i, 0)),
            pl.BlockSpec((f, hh), lambda i: (0, 0)),
            pl.BlockSpec((f, hh), lambda i: (0, 0)),
            pl.BlockSpec((hh,), lambda i: (0,)),
            pl.BlockSpec((f, 1), lambda i: (0, 0)),
            pl.BlockSpec((f, 1), lambda i: (0, 0)),
            pl.BlockSpec((1,), lambda i: (0,)),
            pl.BlockSpec((1,), lambda i: (0,)),
        ],
        out_specs=[
            pl.BlockSpec((_ROW_BLK, hh), lambda i: (i, 0)),
            pl.BlockSpec((_ROW_BLK, hh), lambda i: (i, 0)),
            pl.BlockSpec((_ROW_BLK, 1), lambda i: (i, 0)),
        ],
        out_shape=[
            jax.ShapeDtypeStruct((n, hh), jnp.float32),
            jax.ShapeDtypeStruct((n, hh), jnp.float32),
            jax.ShapeDtypeStruct((n, 1), jnp.float32),
        ],
    )(fea, W_in, W_in_self, b_in, scores0, Dk0, bias0, Dbias0)


def _tc_layer2(parts, dense, W_out, W_out_self, b_out, scores0, Dk0, bias0,
               Dbias0):
    _, n, hh = parts.shape
    cc = W_out.shape[1]
    grid = (n // _ROW_BLK,)

    def body(p_ref, d_ref, wo_ref, wos_ref, bo_ref, sc_ref, dk_ref, b0_ref,
             db_ref, s_out, d_out, sig_out):
        x = p_ref[0] + p_ref[1] + d_ref[...]
        S = jnp.dot(x, wo_ref[...], preferred_element_type=jnp.float32)
        sid = _sigmoid(
            jnp.dot(x, sc_ref[...], preferred_element_type=jnp.float32)
            + b0_ref[...])
        dk = jnp.dot(x, dk_ref[...], preferred_element_type=jnp.float32) \
            + db_ref[...]
        self_t = jnp.dot(x, wos_ref[...],
                         preferred_element_type=jnp.float32) + bo_ref[...]
        D = self_t + _GAMMA * dk * (S + self_t)
        s_out[...] = S
        d_out[...] = D
        sig_out[...] = sid

    return pl.pallas_call(
        body,
        grid=grid,
        in_specs=[
            pl.BlockSpec((2, _ROW_BLK, hh), lambda i: (0, i, 0)),
            pl.BlockSpec((_ROW_BLK, hh), lambda i: (i, 0)),
            pl.BlockSpec((hh, cc), lambda i: (0, 0)),
            pl.BlockSpec((hh, cc), lambda i: (0, 0)),
            pl.BlockSpec((cc,), lambda i: (0,)),
            pl.BlockSpec((hh, 1), lambda i: (0, 0)),
            pl.BlockSpec((hh, 1), lambda i: (0, 0)),
            pl.BlockSpec((1,), lambda i: (0,)),
            pl.BlockSpec((1,), lambda i: (0,)),
        ],
        out_specs=[
            pl.BlockSpec((_ROW_BLK, cc), lambda i: (i, 0)),
            pl.BlockSpec((_ROW_BLK, cc), lambda i: (i, 0)),
            pl.BlockSpec((_ROW_BLK, 1), lambda i: (i, 0)),
        ],
        out_shape=[
            jax.ShapeDtypeStruct((n, cc), jnp.float32),
            jax.ShapeDtypeStruct((n, cc), jnp.float32),
            jax.ShapeDtypeStruct((n, 1), jnp.float32),
        ],
    )(parts, dense, W_out, W_out_self, b_out, scores0, Dk0, bias0, Dbias0)


def _tc_final(parts, dense):
    _, n, cc = parts.shape
    grid = (n // _ROW_BLK,)

    def body(p_ref, d_ref, o_ref):
        z = p_ref[0] + p_ref[1] + d_ref[...]
        m = jnp.max(z, axis=1, keepdims=True)
        zm = z - m
        o_ref[...] = zm - jnp.log(jnp.sum(jnp.exp(zm), axis=1, keepdims=True))

    return pl.pallas_call(
        body,
        grid=grid,
        in_specs=[pl.BlockSpec((2, _ROW_BLK, cc), lambda i: (0, i, 0)),
                  pl.BlockSpec((_ROW_BLK, cc), lambda i: (i, 0))],
        out_specs=pl.BlockSpec((_ROW_BLK, cc), lambda i: (i, 0)),
        out_shape=jax.ShapeDtypeStruct((n, cc), jnp.float32),
    )(parts, dense)


_FAST_SHARE = 0.5


def _pack_rows(index, weight, lo, hi):
    e = hi - lo
    unit = 16 * _B
    epad = ((e + unit - 1) // unit) * unit
    pad = epad - e
    src = jnp.concatenate([index[0][lo:hi], jnp.zeros((pad,), jnp.int32)])
    dst = jnp.concatenate([index[1][lo:hi], jnp.zeros((pad,), jnp.int32)])
    w = jnp.concatenate([weight[lo:hi], jnp.zeros((pad,), jnp.float32)])
    wi = lax.bitcast_convert_type(w, jnp.int32)
    nbw = epad // unit
    comb = jnp.stack([x.reshape(16, nbw, _B) for x in (src, dst, wi)],
                     axis=2)
    return comb, nbw


def _split_pack(index, weight):
    e = weight.shape[0]
    unit = 16 * _B
    ef = int(e * _FAST_SHARE) // unit * unit
    cf, bf = _pack_rows(index, weight, 0, ef)
    cs, bs = _pack_rows(index, weight, ef, e)
    return cf, bf, cs, bs


def _even4(x):
    return ((x + _CH - 1) // _CH) * _CH


def kernel(fea, adj_index, adj_weight, adj_knn_index, adj_knn_weight,
           W_in, W_in_self, b_in, W_out, W_out_self, b_out,
           scores0, bias0, Dk0, Dbias0):
    n_real = fea.shape[0]
    n = _N_PAD
    fea = jnp.pad(fea, ((0, n - n_real), (0, 0)))
    hh = W_in.shape[1]
    cc = W_out.shape[1]

    caf, baf, cas, bas = _split_pack(adj_index, adj_weight)
    ckf, bkf, cks, bks = _split_pack(adj_knn_index, adj_knn_weight)
    nb_f, nb_s = _even4(baf + bkf), _even4(bas + bks)
    nb_max = max(nb_f, nb_s)

    def _rows(ca, ck, used):
        pads = ([jnp.zeros((16, nb_max - used, 3, _B), jnp.int32)]
                if nb_max > used else [])
        return jnp.concatenate([ca, ck] + pads, axis=1)

    comb = jnp.concatenate(
        [_rows(caf, ckf, baf + bkf), _rows(cas, cks, bas + bks)], axis=0)

    S1, D1, sig1 = _tc_layer1(fea, W_in, W_in_self, b_in, scores0, Dk0,
                              bias0, Dbias0)
    sc1 = _make_sc_spmm(n, hh, baf, nb_f, bas, nb_s, nb_max)
    parts1 = sc1(sig1.reshape(n), S1, comb)

    S2, D2, sig2 = _tc_layer2(parts1, D1, W_out, W_out_self, b_out, scores0,
                              Dk0, bias0, Dbias0)
    sc2 = _make_sc_spmm(n, cc, baf, nb_f, bas, nb_s, nb_max)
    parts2 = sc2(sig2.reshape(n), S2, comb)

    return _tc_final(parts2, D2)[:n_real]
